# baseline (speedup 1.0000x reference)
import functools

import jax
import jax.numpy as jnp
from jax import lax
from jax.experimental import pallas as pl
from jax.experimental.pallas import tpu as pltpu
from jax.experimental.pallas import tpu_sc as plsc

N_USERS = 25000
N_ITEMS = 25000
N = N_USERS + N_ITEMS
D = 64
H = D // 2
L = 3
E = 800000
B = 4096

_BLK = 2000

_NS = 16
_NB = 6
_EROWS = 6432
_EPAD = _EROWS * 128
_RPT = _EROWS // _NS
_OROWS = 50176
_STRIPE = _OROWS // _NS


def _spmm_body(edg_hbm, val_hbm, emb_hbm, out_hbm,
               ib0, ib1, ib2, ib3, ib4, ib5,
               vb0, vb1, vb2, vb3, vb4, vb5,
               rb0, rb1, rb2, rb3, rb4, rb5,
               acc, is0, is1, is2, is3, is4, is5,
               gs0, gs1, gs2, gs3, gs4, gs5, ssem):
    cid = lax.axis_index("c")
    sid = lax.axis_index("s")
    ibs = (ib0, ib1, ib2, ib3, ib4, ib5)
    vbs = (vb0, vb1, vb2, vb3, vb4, vb5)
    rbs = (rb0, rb1, rb2, rb3, rb4, rb5)
    isems = (is0, is1, is2, is3, is4, is5)
    gsems = (gs0, gs1, gs2, gs3, gs4, gs5)

    zero16 = jnp.zeros((16,), jnp.float32)

    @pl.loop(0, 64)
    def _zero_buf(k):
        rb0[k, pl.ds(0, 16)] = zero16
        rb0[k, pl.ds(16, 16)] = zero16

    @pl.loop(0, _STRIPE // 64)
    def _zero_acc(z):
        pltpu.sync_copy(rb0.at[pl.ds(0, 64)],
                        acc.at[pl.ds(sid * _STRIPE + z * 64, 64)])

    plsc.subcore_barrier()

    @pl.loop(0, _RPT // _NB)
    def _outer(o):
        s0 = sid * _RPT + o * _NB
        for half in range(2):
            S = (0, 1, 2) if half == 0 else (3, 4, 5)

            @pl.when(o > 0)
            def _drain(S=S):
                for b in S:
                    pltpu.make_async_copy(
                        emb_hbm.at[cid].at[pl.ds(0, 128)], rbs[b], ssem).wait()

            icps = []
            for b in S:
                row = s0 + b
                icps.append((pltpu.async_copy(edg_hbm.at[row], ibs[b], isems[b]),
                             pltpu.async_copy(val_hbm.at[row], vbs[b], isems[b])))
            gcps = []
            for k, b in enumerate(S):
                icps[k][0].wait()
                icps[k][1].wait()
                gcps.append(pltpu.async_copy(emb_hbm.at[cid].at[ibs[b].at[0]],
                                             rbs[b], gsems[b]))
            for k, b in enumerate(S):
                gcps[k].wait()

                @pl.loop(0, 8, unroll=4)
                def _grp(p, b=b):
                    vgrp = vbs[b][pl.ds(p * 16, 16)]
                    for q in range(16):
                        v = vgrp[q]
                        e = p * 16 + q
                        rbs[b][e, pl.ds(0, 16)] = rbs[b][e, pl.ds(0, 16)] * v
                        rbs[b][e, pl.ds(16, 16)] = rbs[b][e, pl.ds(16, 16)] * v

                pltpu.async_copy(rbs[b], acc.at[ibs[b].at[1]], ssem, add=True)

    for b in range(_NB):
        pltpu.make_async_copy(emb_hbm.at[cid].at[pl.ds(0, 128)],
                              rbs[b], ssem).wait()

    plsc.subcore_barrier()
    pltpu.sync_copy(acc.at[pl.ds(sid * _STRIPE, _STRIPE)],
                    out_hbm.at[cid, pl.ds(sid * _STRIPE, _STRIPE)])


_spmm_call = functools.partial(
    pl.kernel,
    out_type=jax.ShapeDtypeStruct((2, _OROWS, H), jnp.float32),
    mesh=plsc.VectorSubcoreMesh(core_axis_name="c", subcore_axis_name="s"),
    scratch_types=(
        [pltpu.VMEM((2, 128), jnp.int32) for _ in range(_NB)]
        + [pltpu.VMEM((128,), jnp.float32) for _ in range(_NB)]
        + [pltpu.VMEM((128, H), jnp.float32) for _ in range(_NB)]
        + [pltpu.VMEM_SHARED((_OROWS, H), jnp.float32)]
        + [pltpu.SemaphoreType.DMA] * (2 * _NB + 1)
    ),
    compiler_params=pltpu.CompilerParams(use_tc_tiling_on_sc=False),
)(_spmm_body)


def _bgather_body(uix, iix, jix, t0, t1, t2, t3, out_hbm, ib, rb, gsem):
    cid = lax.axis_index("c")
    sid = lax.axis_index("s")
    w = sid * 2 + cid
    for s, idx_hbm in enumerate((uix, iix, jix)):
        pltpu.sync_copy(idx_hbm.at[w], ib)
        if s > 0:
            for p in range(8):
                sl = pl.ds(p * 16, 16)
                ib[sl] = ib[sl] + N_USERS
        for t, tab in enumerate((t0, t1, t2, t3)):
            pltpu.async_copy(tab.at[ib], rb, gsem).wait()
            pltpu.sync_copy(rb, out_hbm.at[s, t, pl.ds(w * 128, 128)])


_bgather_call = functools.partial(
    pl.kernel,
    out_type=jax.ShapeDtypeStruct((3, 4, B, D), jnp.float32),
    mesh=plsc.VectorSubcoreMesh(core_axis_name="c", subcore_axis_name="s"),
    scratch_types=[
        pltpu.VMEM((128,), jnp.int32),
        pltpu.VMEM((128, D), jnp.float32),
        pltpu.SemaphoreType.DMA,
    ],
    compiler_params=pltpu.CompilerParams(use_tc_tiling_on_sc=False),
)(_bgather_body)


def _dense_body(e2_ref, sp_ref, w1l_ref, w1h_ref, b1_ref,
                w2l_ref, w2h_ref, b2_ref, norm_ref, e2o_ref):
    el = e2_ref[0]
    eh = e2_ref[1]
    sl_ = sp_ref[0]
    sh_ = sp_ref[1]
    dim = (((1,), (1,)), ((), ()))
    t1 = (lax.dot_general(sl_, w1l_ref[...], dim, preferred_element_type=jnp.float32)
          + lax.dot_general(sh_, w1h_ref[...], dim, preferred_element_type=jnp.float32)
          + b1_ref[...])
    t2 = (lax.dot_general(el * sl_, w2l_ref[...], dim, preferred_element_type=jnp.float32)
          + lax.dot_general(eh * sh_, w2h_ref[...], dim, preferred_element_type=jnp.float32)
          + b2_ref[...])
    x = t1 + t2
    x = jnp.where(x > 0, x, 0.01 * x)
    n = jnp.sqrt(jnp.sum(x * x, axis=1, keepdims=True))
    norm_ref[...] = x / jnp.maximum(n, 1e-12)
    e2o_ref[0] = x[:, :H]
    e2o_ref[1] = x[:, H:]


def _dense_layer(e2, sp2, w1, b1, w2, b2):
    grid = (N // _BLK,)
    return pl.pallas_call(
        _dense_body,
        grid=grid,
        in_specs=[
            pl.BlockSpec((2, _BLK, H), lambda i: (0, i, 0)),
            pl.BlockSpec((2, _BLK, H), lambda i: (0, i, 0)),
            pl.BlockSpec((D, H), lambda i: (0, 0)),
            pl.BlockSpec((D, H), lambda i: (0, 0)),
            pl.BlockSpec((1, D), lambda i: (0, 0)),
            pl.BlockSpec((D, H), lambda i: (0, 0)),
            pl.BlockSpec((D, H), lambda i: (0, 0)),
            pl.BlockSpec((1, D), lambda i: (0, 0)),
        ],
        out_specs=[
            pl.BlockSpec((_BLK, D), lambda i: (i, 0)),
            pl.BlockSpec((2, _BLK, H), lambda i: (0, i, 0)),
        ],
        out_shape=[
            jax.ShapeDtypeStruct((N, D), jnp.float32),
            jax.ShapeDtypeStruct((2, N, H), jnp.float32),
        ],
    )(e2, sp2, w1[:, :H], w1[:, H:], b1.reshape(1, D),
      w2[:, :H], w2[:, H:], b2.reshape(1, D))


def _loss_body(g_ref, out_ref):
    bu = g_ref[0]
    bp = g_ref[1]
    bn = g_ref[2]
    pos = jnp.sum(bu * bp, axis=(0, 2))
    neg = jnp.sum(bu * bn, axis=(0, 2))
    x = pos - neg
    out_ref[0, 0] = -jnp.mean(jnp.log(jax.nn.sigmoid(x)))


def _loss(g):
    return pl.pallas_call(
        _loss_body,
        out_shape=jax.ShapeDtypeStruct((1, 1), jnp.float32),
        out_specs=pl.BlockSpec(memory_space=pltpu.SMEM),
    )(g)[0, 0]


def kernel(u, i, j, edge_index, edge_vals, user_emb, item_emb, W1_w, W1_b, W2_w, W2_b):
    emb = jnp.concatenate((user_emb, item_emb), axis=0)
    e2 = jnp.stack([emb[:, :H], emb[:, H:]])
    pad = _EPAD - E
    src = jnp.concatenate([edge_index[1].astype(jnp.int32),
                           jnp.zeros((pad,), jnp.int32)]).reshape(_EROWS, 128)
    dst = jnp.concatenate([edge_index[0].astype(jnp.int32),
                           jnp.zeros((pad,), jnp.int32)]).reshape(_EROWS, 128)
    val = jnp.concatenate([edge_vals.astype(jnp.float32),
                           jnp.zeros((pad,), jnp.float32)]).reshape(_EROWS, 128)
    edges = jnp.stack([src, dst], axis=1)

    finals = [emb]
    for l in range(L):
        sp2 = _spmm_call(edges, val, e2)
        norm, e2 = _dense_layer(e2, sp2, W1_w[l], W1_b[l], W2_w[l], W2_b[l])
        finals.append(norm)

    uix = u.astype(jnp.int32).reshape(32, 128)
    iix = i.astype(jnp.int32).reshape(32, 128)
    jix = j.astype(jnp.int32).reshape(32, 128)
    g = _bgather_call(uix, iix, jix, finals[0], finals[1], finals[2], finals[3])
    return _loss(g)


# speedup vs baseline: 1.0851x; 1.0851x over previous
import functools

import jax
import jax.numpy as jnp
from jax import lax
from jax.experimental import pallas as pl
from jax.experimental.pallas import tpu as pltpu
from jax.experimental.pallas import tpu_sc as plsc

N_USERS = 25000
N_ITEMS = 25000
N = N_USERS + N_ITEMS
D = 64
H = D // 2
L = 3
E = 800000
B = 4096

_BLK = 2000

_NS = 16
_NB = 6
_EROWS = 6432
_EPAD = _EROWS * 128
_RPT = _EROWS // _NS
_OROWS = 50176
_STRIPE = _OROWS // _NS


def _spmm_body(edg_hbm, val_hbm, emb_hbm, out_hbm,
               ib0, ib1, ib2, ib3, ib4, ib5,
               vb0, vb1, vb2, vb3, vb4, vb5,
               rb0, rb1, rb2, rb3, rb4, rb5,
               acc, is0, is1, is2, is3, is4, is5,
               gs0, gs1, gs2, gs3, gs4, gs5, ssem):
    cid = lax.axis_index("c")
    sid = lax.axis_index("s")
    ibs = (ib0, ib1, ib2, ib3, ib4, ib5)
    vbs = (vb0, vb1, vb2, vb3, vb4, vb5)
    rbs = (rb0, rb1, rb2, rb3, rb4, rb5)
    isems = (is0, is1, is2, is3, is4, is5)
    gsems = (gs0, gs1, gs2, gs3, gs4, gs5)

    zero16 = jnp.zeros((16,), jnp.float32)

    @pl.loop(0, 64)
    def _zero_buf(k):
        rb0[k, pl.ds(0, 16)] = zero16
        rb0[k, pl.ds(16, 16)] = zero16

    @pl.loop(0, _STRIPE // 64)
    def _zero_acc(z):
        pltpu.sync_copy(rb0.at[pl.ds(0, 64)],
                        acc.at[pl.ds(sid * _STRIPE + z * 64, 64)])

    plsc.subcore_barrier()

    @pl.loop(0, _RPT // _NB)
    def _outer(o):
        s0 = sid * _RPT + o * _NB
        icps = []
        for b in range(_NB):
            row = s0 + b
            icps.append((pltpu.async_copy(edg_hbm.at[row], ibs[b], isems[b]),
                         pltpu.async_copy(val_hbm.at[row], vbs[b], isems[b])))
        gcps = []
        for b in range(_NB):
            icps[b][0].wait()
            icps[b][1].wait()
            gcps.append((
                pltpu.async_copy(emb_hbm.at[cid].at[ibs[b].at[0, pl.ds(0, 64)]],
                                 rbs[b].at[pl.ds(0, 64)], gsems[b]),
                pltpu.async_copy(emb_hbm.at[cid].at[ibs[b].at[0, pl.ds(64, 64)]],
                                 rbs[b].at[pl.ds(64, 64)], gsems[b]),
            ))
        scps = []
        for b in range(_NB):
            gcps[b][0].wait()
            gcps[b][1].wait()

            @pl.loop(0, 8, unroll=4)
            def _grp(p, b=b):
                vgrp = vbs[b][pl.ds(p * 16, 16)]
                for q in range(16):
                    v = vgrp[q]
                    e = p * 16 + q
                    rbs[b][e, pl.ds(0, 16)] = rbs[b][e, pl.ds(0, 16)] * v
                    rbs[b][e, pl.ds(16, 16)] = rbs[b][e, pl.ds(16, 16)] * v

            scps.append(pltpu.async_copy(rbs[b], acc.at[ibs[b].at[1]],
                                         ssem, add=True))
        for cp in scps:
            cp.wait()

    plsc.subcore_barrier()
    pltpu.sync_copy(acc.at[pl.ds(sid * _STRIPE, _STRIPE)],
                    out_hbm.at[cid, pl.ds(sid * _STRIPE, _STRIPE)])


_spmm_call = functools.partial(
    pl.kernel,
    out_type=jax.ShapeDtypeStruct((2, _OROWS, H), jnp.float32),
    mesh=plsc.VectorSubcoreMesh(core_axis_name="c", subcore_axis_name="s"),
    scratch_types=(
        [pltpu.VMEM((2, 128), jnp.int32) for _ in range(_NB)]
        + [pltpu.VMEM((128,), jnp.float32) for _ in range(_NB)]
        + [pltpu.VMEM((128, H), jnp.float32) for _ in range(_NB)]
        + [pltpu.VMEM_SHARED((_OROWS, H), jnp.float32)]
        + [pltpu.SemaphoreType.DMA] * (2 * _NB + 1)
    ),
    compiler_params=pltpu.CompilerParams(use_tc_tiling_on_sc=False),
)(_spmm_body)


def _bgather_body(uix, iix, jix, t0, t1, t2, t3, out_hbm, ib, rb, gsem):
    cid = lax.axis_index("c")
    sid = lax.axis_index("s")
    w = sid * 2 + cid
    for s, idx_hbm in enumerate((uix, iix, jix)):
        pltpu.sync_copy(idx_hbm.at[w], ib)
        if s > 0:
            for p in range(8):
                sl = pl.ds(p * 16, 16)
                ib[sl] = ib[sl] + N_USERS
        for t, tab in enumerate((t0, t1, t2, t3)):
            pltpu.async_copy(tab.at[ib], rb, gsem).wait()
            pltpu.sync_copy(rb, out_hbm.at[s, t, pl.ds(w * 128, 128)])


_bgather_call = functools.partial(
    pl.kernel,
    out_type=jax.ShapeDtypeStruct((3, 4, B, D), jnp.float32),
    mesh=plsc.VectorSubcoreMesh(core_axis_name="c", subcore_axis_name="s"),
    scratch_types=[
        pltpu.VMEM((128,), jnp.int32),
        pltpu.VMEM((128, D), jnp.float32),
        pltpu.SemaphoreType.DMA,
    ],
    compiler_params=pltpu.CompilerParams(use_tc_tiling_on_sc=False),
)(_bgather_body)


def _dense_body(e2_ref, sp_ref, w1l_ref, w1h_ref, b1_ref,
                w2l_ref, w2h_ref, b2_ref, norm_ref, e2o_ref):
    el = e2_ref[0]
    eh = e2_ref[1]
    sl_ = sp_ref[0]
    sh_ = sp_ref[1]
    dim = (((1,), (1,)), ((), ()))
    t1 = (lax.dot_general(sl_, w1l_ref[...], dim, preferred_element_type=jnp.float32)
          + lax.dot_general(sh_, w1h_ref[...], dim, preferred_element_type=jnp.float32)
          + b1_ref[...])
    t2 = (lax.dot_general(el * sl_, w2l_ref[...], dim, preferred_element_type=jnp.float32)
          + lax.dot_general(eh * sh_, w2h_ref[...], dim, preferred_element_type=jnp.float32)
          + b2_ref[...])
    x = t1 + t2
    x = jnp.where(x > 0, x, 0.01 * x)
    n = jnp.sqrt(jnp.sum(x * x, axis=1, keepdims=True))
    norm_ref[...] = x / jnp.maximum(n, 1e-12)
    e2o_ref[0] = x[:, :H]
    e2o_ref[1] = x[:, H:]


def _dense_layer(e2, sp2, w1, b1, w2, b2):
    grid = (N // _BLK,)
    return pl.pallas_call(
        _dense_body,
        grid=grid,
        in_specs=[
            pl.BlockSpec((2, _BLK, H), lambda i: (0, i, 0)),
            pl.BlockSpec((2, _BLK, H), lambda i: (0, i, 0)),
            pl.BlockSpec((D, H), lambda i: (0, 0)),
            pl.BlockSpec((D, H), lambda i: (0, 0)),
            pl.BlockSpec((1, D), lambda i: (0, 0)),
            pl.BlockSpec((D, H), lambda i: (0, 0)),
            pl.BlockSpec((D, H), lambda i: (0, 0)),
            pl.BlockSpec((1, D), lambda i: (0, 0)),
        ],
        out_specs=[
            pl.BlockSpec((_BLK, D), lambda i: (i, 0)),
            pl.BlockSpec((2, _BLK, H), lambda i: (0, i, 0)),
        ],
        out_shape=[
            jax.ShapeDtypeStruct((N, D), jnp.float32),
            jax.ShapeDtypeStruct((2, N, H), jnp.float32),
        ],
    )(e2, sp2, w1[:, :H], w1[:, H:], b1.reshape(1, D),
      w2[:, :H], w2[:, H:], b2.reshape(1, D))


def _loss_body(g_ref, out_ref):
    bu = g_ref[0]
    bp = g_ref[1]
    bn = g_ref[2]
    pos = jnp.sum(bu * bp, axis=(0, 2))
    neg = jnp.sum(bu * bn, axis=(0, 2))
    x = pos - neg
    out_ref[0, 0] = -jnp.mean(jnp.log(jax.nn.sigmoid(x)))


def _loss(g):
    return pl.pallas_call(
        _loss_body,
        out_shape=jax.ShapeDtypeStruct((1, 1), jnp.float32),
        out_specs=pl.BlockSpec(memory_space=pltpu.SMEM),
    )(g)[0, 0]


def kernel(u, i, j, edge_index, edge_vals, user_emb, item_emb, W1_w, W1_b, W2_w, W2_b):
    emb = jnp.concatenate((user_emb, item_emb), axis=0)
    e2 = jnp.stack([emb[:, :H], emb[:, H:]])
    pad = _EPAD - E
    src = jnp.concatenate([edge_index[1].astype(jnp.int32),
                           jnp.zeros((pad,), jnp.int32)]).reshape(_EROWS, 128)
    dst = jnp.concatenate([edge_index[0].astype(jnp.int32),
                           jnp.zeros((pad,), jnp.int32)]).reshape(_EROWS, 128)
    val = jnp.concatenate([edge_vals.astype(jnp.float32),
                           jnp.zeros((pad,), jnp.float32)]).reshape(_EROWS, 128)
    edges = jnp.stack([src, dst], axis=1)

    finals = [emb]
    for l in range(L):
        sp2 = _spmm_call(edges, val, e2)
        norm, e2 = _dense_layer(e2, sp2, W1_w[l], W1_b[l], W2_w[l], W2_b[l])
        finals.append(norm)

    uix = u.astype(jnp.int32).reshape(32, 128)
    iix = i.astype(jnp.int32).reshape(32, 128)
    jix = j.astype(jnp.int32).reshape(32, 128)
    g = _bgather_call(uix, iix, jix, finals[0], finals[1], finals[2], finals[3])
    return _loss(g)


# speedup vs baseline: 1.0875x; 1.0022x over previous
import functools

import jax
import jax.numpy as jnp
from jax import lax
from jax.experimental import pallas as pl
from jax.experimental.pallas import tpu as pltpu
from jax.experimental.pallas import tpu_sc as plsc

N_USERS = 25000
N_ITEMS = 25000
N = N_USERS + N_ITEMS
D = 64
H = D // 2
L = 3
E = 800000
B = 4096

_BLK = 2000

_NS = 16
_NB = 6
_EROWS = 6432
_EPAD = _EROWS * 128
_RPT = _EROWS // _NS
_OROWS = 50176
_STRIPE = _OROWS // _NS


def _spmm_body(edg_hbm, val_hbm, emb_hbm, out_hbm,
               ib0, ib1, ib2, ib3, ib4, ib5,
               vb0, vb1, vb2, vb3, vb4, vb5,
               rb0, rb1, rb2, rb3, rb4, rb5,
               acc, is0, is1, is2, is3, is4, is5,
               gs0, gs1, gs2, gs3, gs4, gs5, ssem):
    cid = lax.axis_index("c")
    sid = lax.axis_index("s")
    ibs = (ib0, ib1, ib2, ib3, ib4, ib5)
    vbs = (vb0, vb1, vb2, vb3, vb4, vb5)
    rbs = (rb0, rb1, rb2, rb3, rb4, rb5)
    isems = (is0, is1, is2, is3, is4, is5)
    gsems = (gs0, gs1, gs2, gs3, gs4, gs5)

    zero16 = jnp.zeros((16,), jnp.float32)

    @pl.loop(0, 64)
    def _zero_buf(k):
        rb0[k, pl.ds(0, 16)] = zero16
        rb0[k, pl.ds(16, 16)] = zero16

    @pl.loop(0, _STRIPE // 64)
    def _zero_acc(z):
        pltpu.sync_copy(rb0.at[pl.ds(0, 64)],
                        acc.at[pl.ds(sid * _STRIPE + z * 64, 64)])

    plsc.subcore_barrier()

    @pl.loop(0, _RPT // _NB)
    def _outer(o):
        s0 = sid * _RPT + o * _NB
        icps = []
        for b in range(_NB):
            row = s0 + b
            icps.append((pltpu.async_copy(edg_hbm.at[row], ibs[b], isems[b]),
                         pltpu.async_copy(val_hbm.at[row], vbs[b], isems[b])))
        off = cid * N
        gcps = []
        for b in range(_NB):
            icps[b][0].wait()
            icps[b][1].wait()
            for p in range(8):
                sl = pl.ds(p * 16, 16)
                ibs[b][0, sl] = ibs[b][0, sl] + off
            gcps.append(pltpu.async_copy(emb_hbm.at[ibs[b].at[0]],
                                         rbs[b], gsems[b]))
        scps = []
        for b in range(_NB):
            gcps[b].wait()

            @pl.loop(0, 8, unroll=4)
            def _grp(p, b=b):
                vgrp = vbs[b][pl.ds(p * 16, 16)]
                for q in range(16):
                    v = vgrp[q]
                    e = p * 16 + q
                    rbs[b][e, pl.ds(0, 16)] = rbs[b][e, pl.ds(0, 16)] * v
                    rbs[b][e, pl.ds(16, 16)] = rbs[b][e, pl.ds(16, 16)] * v

            scps.append(pltpu.async_copy(rbs[b], acc.at[ibs[b].at[1]],
                                         ssem, add=True))
        for cp in scps:
            cp.wait()

    plsc.subcore_barrier()
    pltpu.sync_copy(acc.at[pl.ds(sid * _STRIPE, _STRIPE)],
                    out_hbm.at[cid, pl.ds(sid * _STRIPE, _STRIPE)])


_spmm_call = functools.partial(
    pl.kernel,
    out_type=jax.ShapeDtypeStruct((2, _OROWS, H), jnp.float32),
    mesh=plsc.VectorSubcoreMesh(core_axis_name="c", subcore_axis_name="s"),
    scratch_types=(
        [pltpu.VMEM((2, 128), jnp.int32) for _ in range(_NB)]
        + [pltpu.VMEM((128,), jnp.float32) for _ in range(_NB)]
        + [pltpu.VMEM((128, H), jnp.float32) for _ in range(_NB)]
        + [pltpu.VMEM_SHARED((_OROWS, H), jnp.float32)]
        + [pltpu.SemaphoreType.DMA] * (2 * _NB + 1)
    ),
    compiler_params=pltpu.CompilerParams(use_tc_tiling_on_sc=False),
)(_spmm_body)


def _bgather_body(uix, iix, jix, t0, t1, t2, t3, out_hbm, ib, rb, gsem):
    cid = lax.axis_index("c")
    sid = lax.axis_index("s")
    w = sid * 2 + cid
    for s, idx_hbm in enumerate((uix, iix, jix)):
        pltpu.sync_copy(idx_hbm.at[w], ib)
        if s > 0:
            for p in range(8):
                sl = pl.ds(p * 16, 16)
                ib[sl] = ib[sl] + N_USERS
        for t, tab in enumerate((t0, t1, t2, t3)):
            pltpu.async_copy(tab.at[ib], rb, gsem).wait()
            pltpu.sync_copy(rb, out_hbm.at[s, t, pl.ds(w * 128, 128)])


_bgather_call = functools.partial(
    pl.kernel,
    out_type=jax.ShapeDtypeStruct((3, 4, B, D), jnp.float32),
    mesh=plsc.VectorSubcoreMesh(core_axis_name="c", subcore_axis_name="s"),
    scratch_types=[
        pltpu.VMEM((128,), jnp.int32),
        pltpu.VMEM((128, D), jnp.float32),
        pltpu.SemaphoreType.DMA,
    ],
    compiler_params=pltpu.CompilerParams(use_tc_tiling_on_sc=False),
)(_bgather_body)


def _dense_body(e2_ref, sp_ref, w1l_ref, w1h_ref, b1_ref,
                w2l_ref, w2h_ref, b2_ref, norm_ref, e2o_ref):
    el = e2_ref[0]
    eh = e2_ref[1]
    sl_ = sp_ref[0]
    sh_ = sp_ref[1]
    dim = (((1,), (1,)), ((), ()))
    t1 = (lax.dot_general(sl_, w1l_ref[...], dim, preferred_element_type=jnp.float32)
          + lax.dot_general(sh_, w1h_ref[...], dim, preferred_element_type=jnp.float32)
          + b1_ref[...])
    t2 = (lax.dot_general(el * sl_, w2l_ref[...], dim, preferred_element_type=jnp.float32)
          + lax.dot_general(eh * sh_, w2h_ref[...], dim, preferred_element_type=jnp.float32)
          + b2_ref[...])
    x = t1 + t2
    x = jnp.where(x > 0, x, 0.01 * x)
    n = jnp.sqrt(jnp.sum(x * x, axis=1, keepdims=True))
    norm_ref[...] = x / jnp.maximum(n, 1e-12)
    e2o_ref[0] = x[:, :H]
    e2o_ref[1] = x[:, H:]


def _dense_layer(e2, sp2, w1, b1, w2, b2):
    grid = (N // _BLK,)
    return pl.pallas_call(
        _dense_body,
        grid=grid,
        in_specs=[
            pl.BlockSpec((2, _BLK, H), lambda i: (0, i, 0)),
            pl.BlockSpec((2, _BLK, H), lambda i: (0, i, 0)),
            pl.BlockSpec((D, H), lambda i: (0, 0)),
            pl.BlockSpec((D, H), lambda i: (0, 0)),
            pl.BlockSpec((1, D), lambda i: (0, 0)),
            pl.BlockSpec((D, H), lambda i: (0, 0)),
            pl.BlockSpec((D, H), lambda i: (0, 0)),
            pl.BlockSpec((1, D), lambda i: (0, 0)),
        ],
        out_specs=[
            pl.BlockSpec((_BLK, D), lambda i: (i, 0)),
            pl.BlockSpec((2, _BLK, H), lambda i: (0, i, 0)),
        ],
        out_shape=[
            jax.ShapeDtypeStruct((N, D), jnp.float32),
            jax.ShapeDtypeStruct((2, N, H), jnp.float32),
        ],
    )(e2, sp2, w1[:, :H], w1[:, H:], b1.reshape(1, D),
      w2[:, :H], w2[:, H:], b2.reshape(1, D))


def _loss_body(g_ref, out_ref):
    bu = g_ref[0]
    bp = g_ref[1]
    bn = g_ref[2]
    pos = jnp.sum(bu * bp, axis=(0, 2))
    neg = jnp.sum(bu * bn, axis=(0, 2))
    x = pos - neg
    out_ref[0, 0] = -jnp.mean(jnp.log(jax.nn.sigmoid(x)))


def _loss(g):
    return pl.pallas_call(
        _loss_body,
        out_shape=jax.ShapeDtypeStruct((1, 1), jnp.float32),
        out_specs=pl.BlockSpec(memory_space=pltpu.SMEM),
    )(g)[0, 0]


def kernel(u, i, j, edge_index, edge_vals, user_emb, item_emb, W1_w, W1_b, W2_w, W2_b):
    emb = jnp.concatenate((user_emb, item_emb), axis=0)
    e2 = jnp.stack([emb[:, :H], emb[:, H:]])
    pad = _EPAD - E
    src = jnp.concatenate([edge_index[1].astype(jnp.int32),
                           jnp.zeros((pad,), jnp.int32)]).reshape(_EROWS, 128)
    dst = jnp.concatenate([edge_index[0].astype(jnp.int32),
                           jnp.zeros((pad,), jnp.int32)]).reshape(_EROWS, 128)
    val = jnp.concatenate([edge_vals.astype(jnp.float32),
                           jnp.zeros((pad,), jnp.float32)]).reshape(_EROWS, 128)
    edges = jnp.stack([src, dst], axis=1)

    finals = [emb]
    for l in range(L):
        sp2 = _spmm_call(edges, val, e2.reshape(2 * N, H))
        norm, e2 = _dense_layer(e2, sp2, W1_w[l], W1_b[l], W2_w[l], W2_b[l])
        finals.append(norm)

    uix = u.astype(jnp.int32).reshape(32, 128)
    iix = i.astype(jnp.int32).reshape(32, 128)
    jix = j.astype(jnp.int32).reshape(32, 128)
    g = _bgather_call(uix, iix, jix, finals[0], finals[1], finals[2], finals[3])
    return _loss(g)


# speedup vs baseline: 1.1650x; 1.0713x over previous
import functools

import jax
import jax.numpy as jnp
from jax import lax
from jax.experimental import pallas as pl
from jax.experimental.pallas import tpu as pltpu
from jax.experimental.pallas import tpu_sc as plsc

N_USERS = 25000
N_ITEMS = 25000
N = N_USERS + N_ITEMS
D = 64
H = D // 2
L = 3
E = 800000
B = 4096

_BLK = 2000

_NS = 16
_NB = 4
_EROWS = 6400
_EPAD = _EROWS * 128
_RPT = _EROWS // _NS
_OROWS = 50176
_STRIPE = _OROWS // _NS


def _spmm_body(edg_hbm, val_hbm, emb_hbm, out_hbm,
               ib0, ib1, ib2, ib3,
               vb0, vb1, vb2, vb3,
               rb0, rb1, rb2, rb3,
               acc, is0, is1, is2, is3,
               gs0, gs1, gs2, gs3, ssem):
    cid = lax.axis_index("c")
    sid = lax.axis_index("s")
    ibs = (ib0, ib1, ib2, ib3)
    vbs = (vb0, vb1, vb2, vb3)
    rbs = (rb0, rb1, rb2, rb3)
    isems = (is0, is1, is2, is3)
    gsems = (gs0, gs1, gs2, gs3)

    zero16 = jnp.zeros((16,), jnp.float32)

    @pl.loop(0, 64)
    def _zero_buf(k):
        rb0[k, pl.ds(0, 16)] = zero16
        rb0[k, pl.ds(16, 16)] = zero16

    @pl.loop(0, _STRIPE // 64)
    def _zero_acc(z):
        pltpu.sync_copy(rb0.at[pl.ds(0, 64)],
                        acc.at[pl.ds(sid * _STRIPE + z * 64, 64)])

    plsc.subcore_barrier()

    @pl.loop(0, _RPT // _NB)
    def _outer(o):
        s0 = sid * _RPT + o * _NB
        icps = []
        for b in range(_NB):
            row = s0 + b
            icps.append((pltpu.async_copy(edg_hbm.at[row], ibs[b], isems[b]),
                         pltpu.async_copy(val_hbm.at[row], vbs[b], isems[b])))
        off = cid * N
        gcps = []
        for b in range(_NB):
            icps[b][0].wait()
            icps[b][1].wait()
            for p in range(8):
                sl = pl.ds(p * 16, 16)
                ibs[b][0, sl] = ibs[b][0, sl] + off
            gcps.append(pltpu.async_copy(emb_hbm.at[ibs[b].at[0]],
                                         rbs[b], gsems[b]))
        scps = []
        for b in range(_NB):
            gcps[b].wait()

            @pl.loop(0, 8, unroll=4)
            def _grp(p, b=b):
                vgrp = vbs[b][pl.ds(p * 16, 16)]
                for q in range(16):
                    v = vgrp[q]
                    e = p * 16 + q
                    rbs[b][e, pl.ds(0, 16)] = rbs[b][e, pl.ds(0, 16)] * v
                    rbs[b][e, pl.ds(16, 16)] = rbs[b][e, pl.ds(16, 16)] * v

            scps.append(pltpu.async_copy(rbs[b], acc.at[ibs[b].at[1]],
                                         ssem, add=True))
        for cp in scps:
            cp.wait()

    plsc.subcore_barrier()
    pltpu.sync_copy(acc.at[pl.ds(sid * _STRIPE, _STRIPE)],
                    out_hbm.at[cid, pl.ds(sid * _STRIPE, _STRIPE)])


_spmm_call = functools.partial(
    pl.kernel,
    out_type=jax.ShapeDtypeStruct((2, _OROWS, H), jnp.float32),
    mesh=plsc.VectorSubcoreMesh(core_axis_name="c", subcore_axis_name="s"),
    scratch_types=(
        [pltpu.VMEM((2, 128), jnp.int32) for _ in range(_NB)]
        + [pltpu.VMEM((128,), jnp.float32) for _ in range(_NB)]
        + [pltpu.VMEM((128, H), jnp.float32) for _ in range(_NB)]
        + [pltpu.VMEM_SHARED((_OROWS, H), jnp.float32)]
        + [pltpu.SemaphoreType.DMA] * (2 * _NB + 1)
    ),
    compiler_params=pltpu.CompilerParams(use_tc_tiling_on_sc=False),
)(_spmm_body)


def _bgather_body(uix, iix, jix, t0, t1, t2, t3, out_hbm, ib, rb, gsem):
    cid = lax.axis_index("c")
    sid = lax.axis_index("s")
    w = sid * 2 + cid
    for s, idx_hbm in enumerate((uix, iix, jix)):
        pltpu.sync_copy(idx_hbm.at[w], ib)
        if s > 0:
            for p in range(8):
                sl = pl.ds(p * 16, 16)
                ib[sl] = ib[sl] + N_USERS
        for t, tab in enumerate((t0, t1, t2, t3)):
            pltpu.async_copy(tab.at[ib], rb, gsem).wait()
            pltpu.sync_copy(rb, out_hbm.at[s, t, pl.ds(w * 128, 128)])


_bgather_call = functools.partial(
    pl.kernel,
    out_type=jax.ShapeDtypeStruct((3, 4, B, D), jnp.float32),
    mesh=plsc.VectorSubcoreMesh(core_axis_name="c", subcore_axis_name="s"),
    scratch_types=[
        pltpu.VMEM((128,), jnp.int32),
        pltpu.VMEM((128, D), jnp.float32),
        pltpu.SemaphoreType.DMA,
    ],
    compiler_params=pltpu.CompilerParams(use_tc_tiling_on_sc=False),
)(_bgather_body)


def _dense_body(e2_ref, sp_ref, w1l_ref, w1h_ref, b1_ref,
                w2l_ref, w2h_ref, b2_ref, norm_ref, e2o_ref):
    el = e2_ref[0]
    eh = e2_ref[1]
    sl_ = sp_ref[0]
    sh_ = sp_ref[1]
    dim = (((1,), (1,)), ((), ()))
    t1 = (lax.dot_general(sl_, w1l_ref[...], dim, preferred_element_type=jnp.float32)
          + lax.dot_general(sh_, w1h_ref[...], dim, preferred_element_type=jnp.float32)
          + b1_ref[...])
    t2 = (lax.dot_general(el * sl_, w2l_ref[...], dim, preferred_element_type=jnp.float32)
          + lax.dot_general(eh * sh_, w2h_ref[...], dim, preferred_element_type=jnp.float32)
          + b2_ref[...])
    x = t1 + t2
    x = jnp.where(x > 0, x, 0.01 * x)
    n = jnp.sqrt(jnp.sum(x * x, axis=1, keepdims=True))
    norm_ref[...] = x / jnp.maximum(n, 1e-12)
    e2o_ref[0] = x[:, :H]
    e2o_ref[1] = x[:, H:]


def _dense_layer(e2, sp2, w1, b1, w2, b2):
    grid = (N // _BLK,)
    return pl.pallas_call(
        _dense_body,
        grid=grid,
        in_specs=[
            pl.BlockSpec((2, _BLK, H), lambda i: (0, i, 0)),
            pl.BlockSpec((2, _BLK, H), lambda i: (0, i, 0)),
            pl.BlockSpec((D, H), lambda i: (0, 0)),
            pl.BlockSpec((D, H), lambda i: (0, 0)),
            pl.BlockSpec((1, D), lambda i: (0, 0)),
            pl.BlockSpec((D, H), lambda i: (0, 0)),
            pl.BlockSpec((D, H), lambda i: (0, 0)),
            pl.BlockSpec((1, D), lambda i: (0, 0)),
        ],
        out_specs=[
            pl.BlockSpec((_BLK, D), lambda i: (i, 0)),
            pl.BlockSpec((2, _BLK, H), lambda i: (0, i, 0)),
        ],
        out_shape=[
            jax.ShapeDtypeStruct((N, D), jnp.float32),
            jax.ShapeDtypeStruct((2, N, H), jnp.float32),
        ],
    )(e2, sp2, w1[:, :H], w1[:, H:], b1.reshape(1, D),
      w2[:, :H], w2[:, H:], b2.reshape(1, D))


def _loss_body(g_ref, out_ref):
    bu = g_ref[0]
    bp = g_ref[1]
    bn = g_ref[2]
    pos = jnp.sum(bu * bp, axis=(0, 2))
    neg = jnp.sum(bu * bn, axis=(0, 2))
    x = pos - neg
    out_ref[0, 0] = -jnp.mean(jnp.log(jax.nn.sigmoid(x)))


def _loss(g):
    return pl.pallas_call(
        _loss_body,
        out_shape=jax.ShapeDtypeStruct((1, 1), jnp.float32),
        out_specs=pl.BlockSpec(memory_space=pltpu.SMEM),
    )(g)[0, 0]


def kernel(u, i, j, edge_index, edge_vals, user_emb, item_emb, W1_w, W1_b, W2_w, W2_b):
    emb = jnp.concatenate((user_emb, item_emb), axis=0)
    e2 = jnp.stack([emb[:, :H], emb[:, H:]])
    pad = _EPAD - E
    src = jnp.concatenate([edge_index[1].astype(jnp.int32),
                           jnp.zeros((pad,), jnp.int32)]).reshape(_EROWS, 128)
    dst = jnp.concatenate([edge_index[0].astype(jnp.int32),
                           jnp.zeros((pad,), jnp.int32)]).reshape(_EROWS, 128)
    val = jnp.concatenate([edge_vals.astype(jnp.float32),
                           jnp.zeros((pad,), jnp.float32)]).reshape(_EROWS, 128)
    edges = jnp.stack([src, dst], axis=1)

    finals = [emb]
    for l in range(L):
        sp2 = _spmm_call(edges, val, e2.reshape(2 * N, H))
        norm, e2 = _dense_layer(e2, sp2, W1_w[l], W1_b[l], W2_w[l], W2_b[l])
        finals.append(norm)

    uix = u.astype(jnp.int32).reshape(32, 128)
    iix = i.astype(jnp.int32).reshape(32, 128)
    jix = j.astype(jnp.int32).reshape(32, 128)
    g = _bgather_call(uix, iix, jix, finals[0], finals[1], finals[2], finals[3])
    return _loss(g)


# speedup vs baseline: 1.2086x; 1.0374x over previous
import functools

import jax
import jax.numpy as jnp
from jax import lax
from jax.experimental import pallas as pl
from jax.experimental.pallas import tpu as pltpu
from jax.experimental.pallas import tpu_sc as plsc

N_USERS = 25000
N_ITEMS = 25000
N = N_USERS + N_ITEMS
D = 64
H = D // 2
L = 3
E = 800000
B = 4096

_BLK = 2000

_NS = 16
_NB = 5
_EROWS = 6400
_EPAD = _EROWS * 128
_RPT = _EROWS // _NS
_OROWS = 50176
_STRIPE = _OROWS // _NS


def _spmm_body(edg_hbm, val_hbm, emb_hbm, out_hbm,
               ib0, ib1, ib2, ib3, ib4,
               vb0, vb1, vb2, vb3, vb4,
               rb0, rb1, rb2, rb3, rb4,
               acc, is0, is1, is2, is3, is4,
               gs0, gs1, gs2, gs3, gs4, ssem):
    cid = lax.axis_index("c")
    sid = lax.axis_index("s")
    ibs = (ib0, ib1, ib2, ib3, ib4)
    vbs = (vb0, vb1, vb2, vb3, vb4)
    rbs = (rb0, rb1, rb2, rb3, rb4)
    isems = (is0, is1, is2, is3, is4)
    gsems = (gs0, gs1, gs2, gs3, gs4)

    zero16 = jnp.zeros((16,), jnp.float32)

    @pl.loop(0, 64)
    def _zero_buf(k):
        rb0[k, pl.ds(0, 16)] = zero16
        rb0[k, pl.ds(16, 16)] = zero16

    @pl.loop(0, _STRIPE // 64)
    def _zero_acc(z):
        pltpu.sync_copy(rb0.at[pl.ds(0, 64)],
                        acc.at[pl.ds(sid * _STRIPE + z * 64, 64)])

    plsc.subcore_barrier()

    @pl.loop(0, _RPT // _NB)
    def _outer(o):
        s0 = sid * _RPT + o * _NB
        icps = []
        for b in range(_NB):
            row = s0 + b
            icps.append((pltpu.async_copy(edg_hbm.at[row], ibs[b], isems[b]),
                         pltpu.async_copy(val_hbm.at[row], vbs[b], isems[b])))
        off = cid * N
        gcps = []
        for b in range(_NB):
            icps[b][0].wait()
            icps[b][1].wait()
            for p in range(8):
                sl = pl.ds(p * 16, 16)
                ibs[b][0, sl] = ibs[b][0, sl] + off
            gcps.append(pltpu.async_copy(emb_hbm.at[ibs[b].at[0]],
                                         rbs[b], gsems[b]))
        scps = []
        for b in range(_NB):
            gcps[b].wait()

            @pl.loop(0, 8, unroll=4)
            def _grp(p, b=b):
                vgrp = vbs[b][pl.ds(p * 16, 16)]
                for q in range(16):
                    v = vgrp[q]
                    e = p * 16 + q
                    rbs[b][e, pl.ds(0, 16)] = rbs[b][e, pl.ds(0, 16)] * v
                    rbs[b][e, pl.ds(16, 16)] = rbs[b][e, pl.ds(16, 16)] * v

            scps.append(pltpu.async_copy(rbs[b], acc.at[ibs[b].at[1]],
                                         ssem, add=True))
        for cp in scps:
            cp.wait()

    plsc.subcore_barrier()
    pltpu.sync_copy(acc.at[pl.ds(sid * _STRIPE, _STRIPE)],
                    out_hbm.at[cid, pl.ds(sid * _STRIPE, _STRIPE)])


_spmm_call = functools.partial(
    pl.kernel,
    out_type=jax.ShapeDtypeStruct((2, _OROWS, H), jnp.float32),
    mesh=plsc.VectorSubcoreMesh(core_axis_name="c", subcore_axis_name="s"),
    scratch_types=(
        [pltpu.VMEM((2, 128), jnp.int32) for _ in range(_NB)]
        + [pltpu.VMEM((128,), jnp.float32) for _ in range(_NB)]
        + [pltpu.VMEM((128, H), jnp.float32) for _ in range(_NB)]
        + [pltpu.VMEM_SHARED((_OROWS, H), jnp.float32)]
        + [pltpu.SemaphoreType.DMA] * (2 * _NB + 1)
    ),
    compiler_params=pltpu.CompilerParams(use_tc_tiling_on_sc=False),
)(_spmm_body)


def _bgather_body(uix, iix, jix, t0, t1, t2, t3, out_hbm, ib, rb, gsem):
    cid = lax.axis_index("c")
    sid = lax.axis_index("s")
    w = sid * 2 + cid
    for s, idx_hbm in enumerate((uix, iix, jix)):
        pltpu.sync_copy(idx_hbm.at[w], ib)
        if s > 0:
            for p in range(8):
                sl = pl.ds(p * 16, 16)
                ib[sl] = ib[sl] + N_USERS
        for t, tab in enumerate((t0, t1, t2, t3)):
            pltpu.async_copy(tab.at[ib], rb, gsem).wait()
            pltpu.sync_copy(rb, out_hbm.at[s, t, pl.ds(w * 128, 128)])


_bgather_call = functools.partial(
    pl.kernel,
    out_type=jax.ShapeDtypeStruct((3, 4, B, D), jnp.float32),
    mesh=plsc.VectorSubcoreMesh(core_axis_name="c", subcore_axis_name="s"),
    scratch_types=[
        pltpu.VMEM((128,), jnp.int32),
        pltpu.VMEM((128, D), jnp.float32),
        pltpu.SemaphoreType.DMA,
    ],
    compiler_params=pltpu.CompilerParams(use_tc_tiling_on_sc=False),
)(_bgather_body)


def _dense_body(e2_ref, sp_ref, w1l_ref, w1h_ref, b1_ref,
                w2l_ref, w2h_ref, b2_ref, norm_ref, e2o_ref):
    el = e2_ref[0]
    eh = e2_ref[1]
    sl_ = sp_ref[0]
    sh_ = sp_ref[1]
    dim = (((1,), (1,)), ((), ()))
    t1 = (lax.dot_general(sl_, w1l_ref[...], dim, preferred_element_type=jnp.float32)
          + lax.dot_general(sh_, w1h_ref[...], dim, preferred_element_type=jnp.float32)
          + b1_ref[...])
    t2 = (lax.dot_general(el * sl_, w2l_ref[...], dim, preferred_element_type=jnp.float32)
          + lax.dot_general(eh * sh_, w2h_ref[...], dim, preferred_element_type=jnp.float32)
          + b2_ref[...])
    x = t1 + t2
    x = jnp.where(x > 0, x, 0.01 * x)
    n = jnp.sqrt(jnp.sum(x * x, axis=1, keepdims=True))
    norm_ref[...] = x / jnp.maximum(n, 1e-12)
    e2o_ref[0] = x[:, :H]
    e2o_ref[1] = x[:, H:]


def _dense_layer(e2, sp2, w1, b1, w2, b2):
    grid = (N // _BLK,)
    return pl.pallas_call(
        _dense_body,
        grid=grid,
        in_specs=[
            pl.BlockSpec((2, _BLK, H), lambda i: (0, i, 0)),
            pl.BlockSpec((2, _BLK, H), lambda i: (0, i, 0)),
            pl.BlockSpec((D, H), lambda i: (0, 0)),
            pl.BlockSpec((D, H), lambda i: (0, 0)),
            pl.BlockSpec((1, D), lambda i: (0, 0)),
            pl.BlockSpec((D, H), lambda i: (0, 0)),
            pl.BlockSpec((D, H), lambda i: (0, 0)),
            pl.BlockSpec((1, D), lambda i: (0, 0)),
        ],
        out_specs=[
            pl.BlockSpec((_BLK, D), lambda i: (i, 0)),
            pl.BlockSpec((2, _BLK, H), lambda i: (0, i, 0)),
        ],
        out_shape=[
            jax.ShapeDtypeStruct((N, D), jnp.float32),
            jax.ShapeDtypeStruct((2, N, H), jnp.float32),
        ],
    )(e2, sp2, w1[:, :H], w1[:, H:], b1.reshape(1, D),
      w2[:, :H], w2[:, H:], b2.reshape(1, D))


def _loss_body(g_ref, out_ref):
    bu = g_ref[0]
    bp = g_ref[1]
    bn = g_ref[2]
    pos = jnp.sum(bu * bp, axis=(0, 2))
    neg = jnp.sum(bu * bn, axis=(0, 2))
    x = pos - neg
    out_ref[0, 0] = -jnp.mean(jnp.log(jax.nn.sigmoid(x)))


def _loss(g):
    return pl.pallas_call(
        _loss_body,
        out_shape=jax.ShapeDtypeStruct((1, 1), jnp.float32),
        out_specs=pl.BlockSpec(memory_space=pltpu.SMEM),
    )(g)[0, 0]


def kernel(u, i, j, edge_index, edge_vals, user_emb, item_emb, W1_w, W1_b, W2_w, W2_b):
    emb = jnp.concatenate((user_emb, item_emb), axis=0)
    e2 = jnp.stack([emb[:, :H], emb[:, H:]])
    pad = _EPAD - E
    src = jnp.concatenate([edge_index[1].astype(jnp.int32),
                           jnp.zeros((pad,), jnp.int32)]).reshape(_EROWS, 128)
    dst = jnp.concatenate([edge_index[0].astype(jnp.int32),
                           jnp.zeros((pad,), jnp.int32)]).reshape(_EROWS, 128)
    val = jnp.concatenate([edge_vals.astype(jnp.float32),
                           jnp.zeros((pad,), jnp.float32)]).reshape(_EROWS, 128)
    edges = jnp.stack([src, dst], axis=1)

    finals = [emb]
    for l in range(L):
        sp2 = _spmm_call(edges, val, e2.reshape(2 * N, H))
        norm, e2 = _dense_layer(e2, sp2, W1_w[l], W1_b[l], W2_w[l], W2_b[l])
        finals.append(norm)

    uix = u.astype(jnp.int32).reshape(32, 128)
    iix = i.astype(jnp.int32).reshape(32, 128)
    jix = j.astype(jnp.int32).reshape(32, 128)
    g = _bgather_call(uix, iix, jix, finals[0], finals[1], finals[2], finals[3])
    return _loss(g)
